# baseline (speedup 1.0000x reference)
import functools

import jax
import jax.numpy as jnp
from jax import lax
from jax.experimental import pallas as pl
from jax.experimental.pallas import tpu as pltpu
from jax.experimental.pallas import tpu_sc as plsc

B, T, D_WP, HID = 16384, 20, 3, 512
BBig = 8192


def _body(wp_ref, wt_ref, pb_ref, emb_ref, out_ref):
    t = pl.program_id(1)
    acc = emb_ref[pl.ds(t, 1), :] + pb_ref[...]
    lhs = wp_ref[:, pl.ds(t, 1), :].reshape(D_WP, out_ref.shape[1])
    prod = jax.lax.dot_general(
        lhs,
        wt_ref[...],
        dimension_numbers=(((0,), (0,)), ((), ())),
        preferred_element_type=jnp.float32,
    )
    out_ref[0] = prod + acc


def _sc_gather(positions, emb_flat):
    @functools.partial(
        pl.kernel,
        out_type=jax.ShapeDtypeStruct((T * HID,), jnp.float32),
        mesh=plsc.VectorSubcoreMesh(core_axis_name="c", subcore_axis_name="s"),
        scratch_types=[
            pltpu.VMEM((T,), jnp.int32),
            pltpu.VMEM((T * HID,), jnp.float32),
        ],
    )
    def k(pos_hbm, emb_hbm, out_hbm, idx_v, rows_v):
        wid = lax.axis_index("s") * 2 + lax.axis_index("c")

        @pl.when(wid == 0)
        def _():
            pltpu.sync_copy(pos_hbm, idx_v)
            for i in range(T):
                pltpu.sync_copy(
                    emb_hbm.at[pl.ds(i * HID, HID)],
                    rows_v.at[pl.ds(i * HID, HID)],
                )
            pltpu.sync_copy(rows_v, out_hbm)

    return k(positions, emb_flat)


@functools.partial(jax.jit)
def kernel(waypoints, proj_w, proj_b, emb_table):
    positions = jnp.arange(T, dtype=jnp.int32)
    comb_rows = _sc_gather(positions, emb_table.reshape(T * HID))
    comb_rows = comb_rows.reshape(T, HID)
    wpP = waypoints.transpose(2, 1, 0)
    wt = proj_w.T
    pb = proj_b.reshape(1, HID)
    out = pl.pallas_call(
        _body,
        grid=(B // BBig, T),
        in_specs=[
            pl.BlockSpec((D_WP, T, BBig), lambda i, t: (0, 0, i)),
            pl.BlockSpec((D_WP, HID), lambda i, t: (0, 0)),
            pl.BlockSpec((1, HID), lambda i, t: (0, 0)),
            pl.BlockSpec((T, HID), lambda i, t: (0, 0)),
        ],
        out_specs=pl.BlockSpec((1, BBig, HID), lambda i, t: (t, i, 0)),
        out_shape=jax.ShapeDtypeStruct((T, B, HID), jnp.float32),
        compiler_params=pltpu.CompilerParams(
            dimension_semantics=("arbitrary", "arbitrary"),
        ),
    )(wpP, wt, pb, comb_rows)
    return out.transpose(1, 0, 2)


# speedup vs baseline: 1.1572x; 1.1572x over previous
import functools

import jax
import jax.numpy as jnp
from jax.experimental import pallas as pl
from jax.experimental.pallas import tpu as pltpu

B, T, D_WP, HID = 16384, 20, 3, 512
BBig = 8192


def _body(wp_ref, wt_ref, pb_ref, emb_ref, out_ref):
    t = pl.program_id(1)
    acc = emb_ref[pl.ds(t, 1), :] + pb_ref[...]
    lhs = wp_ref[:, pl.ds(t, 1), :].reshape(D_WP, out_ref.shape[1])
    prod = jax.lax.dot_general(
        lhs,
        wt_ref[...],
        dimension_numbers=(((0,), (0,)), ((), ())),
        preferred_element_type=jnp.float32,
    )
    out_ref[0] = prod + acc


@functools.partial(jax.jit)
def kernel(waypoints, proj_w, proj_b, emb_table):
    wpP = waypoints.transpose(2, 1, 0)
    wt = proj_w.T
    pb = proj_b.reshape(1, HID)
    out = pl.pallas_call(
        _body,
        grid=(B // BBig, T),
        in_specs=[
            pl.BlockSpec((D_WP, T, BBig), lambda i, t: (0, 0, i)),
            pl.BlockSpec((D_WP, HID), lambda i, t: (0, 0)),
            pl.BlockSpec((1, HID), lambda i, t: (0, 0)),
            pl.BlockSpec((T, HID), lambda i, t: (0, 0)),
        ],
        out_specs=pl.BlockSpec((1, BBig, HID), lambda i, t: (t, i, 0)),
        out_shape=jax.ShapeDtypeStruct((T, B, HID), jnp.float32),
        compiler_params=pltpu.CompilerParams(
            dimension_semantics=("arbitrary", "arbitrary"),
        ),
    )(wpP, wt, pb, emb_table)
    return out.transpose(1, 0, 2)
